# Initial kernel scaffold; baseline (speedup 1.0000x reference)
import jax
import jax.numpy as jnp
from jax.experimental import pallas as pl


def kernel(edge_list, feature_dict, node_type_list, W, b_lin, att_W, att_b, bias):
    raise NotImplementedError("write your pallas kernel here")


# speedup vs baseline: 7.0424x; 7.0424x over previous
import functools

import jax
import jax.numpy as jnp
from jax import lax
from jax.experimental import pallas as pl
from jax.experimental.pallas import tpu as pltpu
from jax.experimental.pallas import tpu_sc as plsc

NC, NS, LANES = 2, 16, 16
NW = NC * NS
CHUNK = 80


def _tc_prep(x_ref, w_ref, b_ref, c_ref, ab_ref, h_ref, s_ref):
    i = pl.program_id(0)
    x = x_ref[0]
    h = jnp.dot(x, w_ref[...].T, preferred_element_type=jnp.float32) + b_ref[...]
    h_ref[0] = h
    s = jnp.dot(h, c_ref[0, 0][:, None], preferred_element_type=jnp.float32)[:, 0]
    s_ref[0, 0] = s + jnp.where(i == 0, ab_ref[0, 0], 0.0)


def _sc_edges(e0_hbm, i1_hbm, i2_hbm, hflat_hbm, sflat_hbm, zrow_hbm,
              part_hbm,
              e0_v, i1_v, i2_v, g0_v, g1_v, g2_v, p_v, buf1, buf2, comb,
              s_sh, acc_sh, sem1, sem2, sem3, sem4, sem5):
    c = lax.axis_index("c")
    s = lax.axis_index("s")
    wid = s * NC + c
    n3, h = hflat_hbm.shape
    n = n3 // 3
    hw = h + LANES
    e = e0_hbm.shape[0]
    epw = e // NW
    nchunk = epw // CHUNK
    zrows = n // 10

    @pl.when(s == 0)
    def _stage_s():
        pltpu.sync_copy(sflat_hbm, s_sh)

    @pl.when(s < 10)
    def _zero_acc():
        pltpu.sync_copy(zrow_hbm, acc_sh.at[pl.ds(s * zrows, zrows)])

    plsc.subcore_barrier()

    def chunk_body(t, carry):
        off = wid * epw + t * CHUNK
        pltpu.sync_copy(e0_hbm.at[pl.ds(off, CHUNK)], e0_v)
        pltpu.sync_copy(i1_hbm.at[pl.ds(off, CHUNK)], i1_v)
        pltpu.sync_copy(i2_hbm.at[pl.ds(off, CHUNK)], i2_v)
        cp1 = pltpu.async_copy(hflat_hbm.at[i1_v], buf1, sem1)
        cp2 = pltpu.async_copy(hflat_hbm.at[i2_v], buf2, sem2)
        cg0 = pltpu.async_copy(s_sh.at[e0_v], g0_v, sem3)
        cg1 = pltpu.async_copy(s_sh.at[i1_v], g1_v, sem4)
        cg2 = pltpu.async_copy(s_sh.at[i2_v], g2_v, sem5)
        cg0.wait()
        cg1.wait()
        cg2.wait()

        def pgrp(j, carry2):
            sl = pl.ds(j * LANES, LANES)
            ee = g0_v[sl] + g1_v[sl] + g2_v[sl]
            ex = jnp.exp(ee + ee)
            th = 1.0 - 2.0 / (ex + 1.0)
            p_v[sl] = jnp.exp(th)
            return carry2

        lax.fori_loop(0, CHUNK // LANES, pgrp, 0)
        cp1.wait()
        cp2.wait()

        onehot0 = jnp.where(lax.iota(jnp.int32, LANES) == 0, 1.0, 0.0)

        def rowgrp(g, carry3):
            base = g * LANES
            pv = p_v[pl.ds(base, LANES)]
            for r in range(LANES):
                pvec = jnp.full((LANES,), pv[r], jnp.float32)
                j = base + r
                for cb in range(h // LANES):
                    sl = pl.ds(cb * LANES, LANES)
                    comb[j, sl] = (buf1[j, sl] + buf2[j, sl]) * pvec
                comb[j, pl.ds(h, LANES)] = pvec * onehot0
            return carry3

        lax.fori_loop(0, CHUNK // LANES, rowgrp, 0)
        pltpu.sync_copy(comb, acc_sh.at[e0_v], add=True)
        return carry

    lax.fori_loop(0, nchunk, chunk_body, 0)
    plsc.subcore_barrier()

    @pl.when(s < 10)
    def _dump_acc():
        pltpu.sync_copy(acc_sh.at[pl.ds(s * zrows, zrows)],
                        part_hbm.at[c].at[pl.ds(s * zrows, zrows)])


def _tc_fin(h_ref, part_ref, bias_ref, o_ref):
    hdim = h_ref.shape[2]
    acc = part_ref[0] + part_ref[1]
    d = acc[:, hdim]
    p = acc[:, :hdim]
    h0 = h_ref[0]
    t = jnp.where(d > 0.0, 1.0, 0.0)
    r = 1.0 / jnp.maximum(d, 1e-16)
    o_ref[...] = (h0 * t[:, None] + p * r[:, None]) * (1.0 / 3.0) + bias_ref[...]


def kernel(edge_list, feature_dict, node_type_list, W, b_lin, att_W, att_b, bias):
    l, n, fin = feature_dict.shape
    hdim = W.shape[0]
    e = edge_list.shape[1]
    f32 = jnp.float32

    a1 = att_W[0, :hdim]
    a2 = att_W[0, hdim:]
    c_all = jnp.stack([a1 + a2 / 3.0, a2 / 3.0, a2 / 3.0], 0)

    h_all, s_all = pl.pallas_call(
        _tc_prep,
        grid=(l,),
        in_specs=[
            pl.BlockSpec((1, n, fin), lambda i: (i, 0, 0)),
            pl.BlockSpec((hdim, fin), lambda i: (0, 0)),
            pl.BlockSpec((1, hdim), lambda i: (0, 0)),
            pl.BlockSpec((1, 1, hdim), lambda i: (i, 0, 0)),
            pl.BlockSpec((1, 1), lambda i: (0, 0)),
        ],
        out_specs=[
            pl.BlockSpec((1, n, hdim), lambda i: (i, 0, 0)),
            pl.BlockSpec((1, 1, n), lambda i: (i, 0, 0)),
        ],
        out_shape=[
            jax.ShapeDtypeStruct((l, n, hdim), f32),
            jax.ShapeDtypeStruct((l, 1, n), f32),
        ],
    )(feature_dict, W, b_lin.reshape(1, hdim), c_all.reshape(l, 1, hdim),
      att_b.reshape(1, 1))

    e0 = edge_list[0]
    i1 = edge_list[1] + n
    i2 = edge_list[2] + 2 * n
    h_flat = h_all.reshape(l * n, hdim)
    s_flat = s_all.reshape(l * n)
    hw = hdim + LANES
    zrow = jnp.zeros((n // 10, hw), f32)

    mesh = plsc.VectorSubcoreMesh(core_axis_name="c", subcore_axis_name="s")
    sc_fn = pl.kernel(
        _sc_edges,
        out_type=jax.ShapeDtypeStruct((NC, n, hw), f32),
        mesh=mesh,
        compiler_params=pltpu.CompilerParams(needs_layout_passes=False,
                                             use_tc_tiling_on_sc=False),
        scratch_types=[
            pltpu.VMEM((CHUNK,), jnp.int32),
            pltpu.VMEM((CHUNK,), jnp.int32),
            pltpu.VMEM((CHUNK,), jnp.int32),
            pltpu.VMEM((CHUNK,), f32),
            pltpu.VMEM((CHUNK,), f32),
            pltpu.VMEM((CHUNK,), f32),
            pltpu.VMEM((CHUNK,), f32),
            pltpu.VMEM((CHUNK, hdim), f32),
            pltpu.VMEM((CHUNK, hdim), f32),
            pltpu.VMEM((CHUNK, hw), f32),
            pltpu.VMEM_SHARED((l * n,), f32),
            pltpu.VMEM_SHARED((n, hw), f32),
            pltpu.SemaphoreType.DMA,
            pltpu.SemaphoreType.DMA,
            pltpu.SemaphoreType.DMA,
            pltpu.SemaphoreType.DMA,
            pltpu.SemaphoreType.DMA,
        ],
    )
    part = sc_fn(e0, i1, i2, h_flat, s_flat, zrow)

    out = pl.pallas_call(
        _tc_fin,
        grid=(1,),
        in_specs=[
            pl.BlockSpec((1, n, hdim), lambda i: (0, 0, 0)),
            pl.BlockSpec((NC, n, hw), lambda i: (0, 0, 0)),
            pl.BlockSpec((1, hdim), lambda i: (0, 0)),
        ],
        out_specs=pl.BlockSpec((n, hdim), lambda i: (0, 0)),
        out_shape=jax.ShapeDtypeStruct((n, hdim), f32),
    )(h_all, part, bias.reshape(1, hdim))
    return out


# speedup vs baseline: 10.8709x; 1.5436x over previous
import functools

import jax
import jax.numpy as jnp
from jax import lax
from jax.experimental import pallas as pl
from jax.experimental.pallas import tpu as pltpu
from jax.experimental.pallas import tpu_sc as plsc

NC, NS, LANES = 2, 16, 16
NW = NC * NS
CHUNK = 128


def _tc_prep(x_ref, w_ref, b_ref, c_ref, ab_ref, h_ref, s_ref):
    i = pl.program_id(0)
    x = x_ref[0]
    h = jnp.dot(x, w_ref[...].T, preferred_element_type=jnp.float32) + b_ref[...]
    h_ref[0] = h
    s = jnp.dot(h, c_ref[0, 0][:, None], preferred_element_type=jnp.float32)[:, 0]
    s_ref[0, 0] = s + jnp.where(i == 0, ab_ref[0, 0], 0.0)


def _sc_edges(e0_hbm, i1_hbm, i2_hbm, hb_hbm, sflat_hbm, zrow_hbm,
              part_hbm,
              e0_v, i1_v, i2_v, g0_v, g1_v, g2_v, p_v, buf1, buf2, comb,
              s_sh, acc_sh,
              se0, se1, se2, sg0, sg1, sg2, sr1, sr2, ssc):
    c = lax.axis_index("c")
    s = lax.axis_index("s")
    wid = s * NC + c
    n3 = sflat_hbm.shape[0]
    n = n3 // 3
    h = hb_hbm.shape[1]
    hw = h + LANES
    e = e0_hbm.shape[0]
    total_chunks = e // CHUNK
    base_chunks = total_chunks // NW
    extra = total_chunks - base_chunks * NW
    nw = base_chunks + jnp.where(wid < extra, 1, 0)
    zrows = n // 10

    @pl.when(s == 0)
    def _stage_s():
        pltpu.sync_copy(sflat_hbm, s_sh)

    @pl.when(s < 10)
    def _zero_acc():
        pltpu.sync_copy(zrow_hbm, acc_sh.at[pl.ds(s * zrows, zrows)])

    plsc.subcore_barrier()

    def fire_idx(chunk_id, slot):
        off = chunk_id * CHUNK
        pltpu.async_copy(e0_hbm.at[pl.ds(off, CHUNK)], e0_v.at[slot], se0)
        pltpu.async_copy(i1_hbm.at[pl.ds(off, CHUNK)], i1_v.at[slot], se1)
        pltpu.async_copy(i2_hbm.at[pl.ds(off, CHUNK)], i2_v.at[slot], se2)

    fire_idx(wid, 0)

    onehot0 = jnp.where(lax.iota(jnp.int32, LANES) == 0, 1.0, 0.0)

    def chunk_body(t, carry):
        slot = lax.rem(t, 2)
        pltpu.make_async_copy(
            e0_hbm.at[pl.ds(0, CHUNK)], e0_v.at[slot], se0).wait()
        pltpu.make_async_copy(
            i1_hbm.at[pl.ds(0, CHUNK)], i1_v.at[slot], se1).wait()
        pltpu.make_async_copy(
            i2_hbm.at[pl.ds(0, CHUNK)], i2_v.at[slot], se2).wait()
        cg0 = pltpu.async_copy(s_sh.at[e0_v.at[slot]], g0_v, sg0)
        cg1 = pltpu.async_copy(s_sh.at[i1_v.at[slot]], g1_v, sg1)
        cg2 = pltpu.async_copy(s_sh.at[i2_v.at[slot]], g2_v, sg2)
        cr1 = pltpu.async_copy(hb_hbm.at[i1_v.at[slot]], buf1, sr1)
        cr2 = pltpu.async_copy(hb_hbm.at[i2_v.at[slot]], buf2, sr2)

        @pl.when(t > 0)
        def _drain_sc():
            pltpu.make_async_copy(comb, acc_sh.at[e0_v.at[1 - slot]],
                                  ssc).wait()

        @pl.when(t + 1 < nw)
        def _prefetch_idx():
            fire_idx(wid + NW * (t + 1), 1 - slot)

        cg0.wait()
        cg1.wait()
        cg2.wait()

        def pgrp(j, carry2):
            sl = pl.ds(j * LANES, LANES)
            ee = g0_v[sl] + g1_v[sl] + g2_v[sl]
            ex = jnp.exp(ee + ee)
            th = 1.0 - 2.0 / (ex + 1.0)
            p_v[sl] = jnp.exp(th)
            return carry2

        lax.fori_loop(0, CHUNK // LANES, pgrp, 0)
        cr1.wait()
        cr2.wait()

        def rowgrp(g, carry3):
            bs = g * LANES
            pv = p_v[pl.ds(bs, LANES)]
            for r in range(LANES):
                pvec = jnp.full((LANES,), pv[r], jnp.float32)
                j = bs + r
                for blk in range(h // 32):
                    ab1 = buf1[j, pl.ds(blk * 32, 32)]
                    ab2 = buf2[j, pl.ds(blk * 32, 32)]
                    a1, b1 = plsc.unpack(ab1, format=plsc.PackFormat.INTERLEAVED)
                    a2, b2 = plsc.unpack(ab2, format=plsc.PackFormat.INTERLEAVED)
                    comb[j, pl.ds(blk * 32, LANES)] = (a1 + a2) * pvec
                    comb[j, pl.ds(blk * 32 + LANES, LANES)] = (b1 + b2) * pvec
                comb[j, pl.ds(h, LANES)] = pvec * onehot0
            return carry3

        lax.fori_loop(0, CHUNK // LANES, rowgrp, 0)
        pltpu.async_copy(comb, acc_sh.at[e0_v.at[slot]], ssc, add=True)
        return carry

    lax.fori_loop(0, nw, chunk_body, 0)
    pltpu.make_async_copy(comb, acc_sh.at[e0_v.at[lax.rem(nw - 1, 2)]],
                          ssc).wait()
    plsc.subcore_barrier()

    @pl.when(s < 10)
    def _dump_acc():
        pltpu.sync_copy(acc_sh.at[pl.ds(s * zrows, zrows)],
                        part_hbm.at[c].at[pl.ds(s * zrows, zrows)])


def _tc_fin(h_ref, part_ref, bias_ref, o_ref):
    hdim = h_ref.shape[2]
    acc = part_ref[0] + part_ref[1]
    d = acc[:, hdim]
    p = acc[:, :hdim]
    h0 = h_ref[0]
    t = jnp.where(d > 0.0, 1.0, 0.0)
    r = 1.0 / jnp.maximum(d, 1e-16)
    o_ref[...] = (h0 * t[:, None] + p * r[:, None]) * (1.0 / 3.0) + bias_ref[...]


def kernel(edge_list, feature_dict, node_type_list, W, b_lin, att_W, att_b, bias):
    l, n, fin = feature_dict.shape
    hdim = W.shape[0]
    e = edge_list.shape[1]
    f32 = jnp.float32

    a1 = att_W[0, :hdim]
    a2 = att_W[0, hdim:]
    c_all = jnp.stack([a1 + a2 / 3.0, a2 / 3.0, a2 / 3.0], 0)

    h_all, s_all = pl.pallas_call(
        _tc_prep,
        grid=(l,),
        in_specs=[
            pl.BlockSpec((1, n, fin), lambda i: (i, 0, 0)),
            pl.BlockSpec((hdim, fin), lambda i: (0, 0)),
            pl.BlockSpec((1, hdim), lambda i: (0, 0)),
            pl.BlockSpec((1, 1, hdim), lambda i: (i, 0, 0)),
            pl.BlockSpec((1, 1), lambda i: (0, 0)),
        ],
        out_specs=[
            pl.BlockSpec((1, n, hdim), lambda i: (i, 0, 0)),
            pl.BlockSpec((1, 1, n), lambda i: (i, 0, 0)),
        ],
        out_shape=[
            jax.ShapeDtypeStruct((l, n, hdim), f32),
            jax.ShapeDtypeStruct((l, 1, n), f32),
        ],
    )(feature_dict, W, b_lin.reshape(1, hdim), c_all.reshape(l, 1, hdim),
      att_b.reshape(1, 1))

    e0 = edge_list[0]
    i1 = edge_list[1] + n
    i2 = edge_list[2] + 2 * n
    hb = (h_all.astype(jnp.bfloat16)
          .reshape(l * n, hdim // 32, 2, LANES)
          .swapaxes(2, 3)
          .reshape(l * n, hdim))
    s_flat = s_all.reshape(l * n)
    hw = hdim + LANES
    zrow = jnp.zeros((n // 10, hw), f32)

    mesh = plsc.VectorSubcoreMesh(core_axis_name="c", subcore_axis_name="s")
    sc_fn = pl.kernel(
        _sc_edges,
        out_type=jax.ShapeDtypeStruct((NC, n, hw), f32),
        mesh=mesh,
        compiler_params=pltpu.CompilerParams(needs_layout_passes=False,
                                             use_tc_tiling_on_sc=False),
        scratch_types=[
            pltpu.VMEM((2, CHUNK), jnp.int32),
            pltpu.VMEM((2, CHUNK), jnp.int32),
            pltpu.VMEM((2, CHUNK), jnp.int32),
            pltpu.VMEM((CHUNK,), f32),
            pltpu.VMEM((CHUNK,), f32),
            pltpu.VMEM((CHUNK,), f32),
            pltpu.VMEM((CHUNK,), f32),
            pltpu.VMEM((CHUNK, hdim), jnp.bfloat16),
            pltpu.VMEM((CHUNK, hdim), jnp.bfloat16),
            pltpu.VMEM((CHUNK, hw), f32),
            pltpu.VMEM_SHARED((l * n,), f32),
            pltpu.VMEM_SHARED((n, hw), f32),
        ] + [pltpu.SemaphoreType.DMA] * 9,
    )
    part = sc_fn(e0, i1, i2, hb, s_flat, zrow)

    out = pl.pallas_call(
        _tc_fin,
        grid=(1,),
        in_specs=[
            pl.BlockSpec((1, n, hdim), lambda i: (0, 0, 0)),
            pl.BlockSpec((NC, n, hw), lambda i: (0, 0, 0)),
            pl.BlockSpec((1, hdim), lambda i: (0, 0)),
        ],
        out_specs=pl.BlockSpec((n, hdim), lambda i: (0, 0)),
        out_shape=jax.ShapeDtypeStruct((n, hdim), f32),
    )(h_all, part, bias.reshape(1, hdim))
    return out


# speedup vs baseline: 14.0501x; 1.2924x over previous
import functools

import jax
import jax.numpy as jnp
from jax import lax
from jax.experimental import pallas as pl
from jax.experimental.pallas import tpu as pltpu
from jax.experimental.pallas import tpu_sc as plsc

NC, NS, LANES = 2, 16, 16
NW = NC * NS
CHUNK = 128


def _tc_prep(x_ref, w_ref, b_ref, c_ref, ab_ref, h_ref, s_ref):
    i = pl.program_id(0)
    x = x_ref[0]
    h = jnp.dot(x, w_ref[...].T, preferred_element_type=jnp.float32) + b_ref[...]
    h_ref[0] = h
    s = jnp.dot(h, c_ref[0, 0][:, None], preferred_element_type=jnp.float32)[:, 0]
    s_ref[0, 0] = s + jnp.where(i == 0, ab_ref[0, 0], 0.0)


def _sc_edges(e0_hbm, i1_hbm, i2_hbm, hb_hbm, sflat_hbm,
              partb_hbm, partd_hbm,
              e0_v, i1_v, i2_v, g0_v, g1_v, g2_v, p_v, buf1, buf2, comb,
              comb_d, s_sh, acc_sh, accd_sh,
              se0, se1, se2, sg0, sg1, sg2, sr1, sr2, ssc, ssd):
    c = lax.axis_index("c")
    s = lax.axis_index("s")
    wid = s * NC + c
    n3 = sflat_hbm.shape[0]
    n = n3 // 3
    h = hb_hbm.shape[1]
    e = e0_hbm.shape[0]
    total_chunks = e // CHUNK
    base_chunks = total_chunks // NW
    extra = total_chunks - base_chunks * NW
    nw = base_chunks + jnp.where(wid < extra, 1, 0)
    zrows = n // 10
    bf16 = jnp.bfloat16

    @pl.when(s == 0)
    def _stage_s():
        pltpu.sync_copy(sflat_hbm, s_sh)

    zb = jnp.zeros((2 * LANES,), bf16)
    zf = jnp.zeros((LANES,), jnp.float32)
    iota = lax.iota(jnp.int32, LANES)

    def _zcomb(j, carry):
        for blk in range(h // (2 * LANES)):
            comb[j, pl.ds(blk * 2 * LANES, 2 * LANES)] = zb
        return carry

    lax.fori_loop(0, CHUNK, _zcomb, 0)

    def _zcombd(j, carry):
        ridx = 2 * j + lax.shift_right_logical(iota, 3)
        cidx = lax.bitwise_and(iota, 7)
        plsc.store_scatter(comb_d, [ridx, cidx], zf)
        return carry

    lax.fori_loop(0, CHUNK // 2, _zcombd, 0)

    zpc = CHUNK - 3

    @pl.when(s < 10)
    def _zero_acc():
        def _zcp(k, carry):
            r0 = s * zrows + k * zpc
            pltpu.sync_copy(comb.at[pl.ds(0, zpc)], acc_sh.at[pl.ds(r0, zpc)])
            pltpu.sync_copy(comb_d.at[pl.ds(0, zpc)], accd_sh.at[pl.ds(r0, zpc)])
            return carry

        lax.fori_loop(0, zrows // zpc, _zcp, 0)

    plsc.subcore_barrier()

    def fire_idx(chunk_id, slot):
        off = chunk_id * CHUNK
        pltpu.async_copy(e0_hbm.at[pl.ds(off, CHUNK)], e0_v.at[slot], se0)
        pltpu.async_copy(i1_hbm.at[pl.ds(off, CHUNK)], i1_v.at[slot], se1)
        pltpu.async_copy(i2_hbm.at[pl.ds(off, CHUNK)], i2_v.at[slot], se2)

    fire_idx(wid, 0)

    def chunk_body(t, carry):
        slot = lax.rem(t, 2)
        pltpu.make_async_copy(
            e0_hbm.at[pl.ds(0, CHUNK)], e0_v.at[slot], se0).wait()
        pltpu.make_async_copy(
            i1_hbm.at[pl.ds(0, CHUNK)], i1_v.at[slot], se1).wait()
        pltpu.make_async_copy(
            i2_hbm.at[pl.ds(0, CHUNK)], i2_v.at[slot], se2).wait()
        cg0 = pltpu.async_copy(s_sh.at[e0_v.at[slot]], g0_v, sg0)
        cg1 = pltpu.async_copy(s_sh.at[i1_v.at[slot]], g1_v, sg1)
        cg2 = pltpu.async_copy(s_sh.at[i2_v.at[slot]], g2_v, sg2)
        cr1 = pltpu.async_copy(hb_hbm.at[i1_v.at[slot]], buf1, sr1)
        cr2 = pltpu.async_copy(hb_hbm.at[i2_v.at[slot]], buf2, sr2)

        @pl.when(t > 0)
        def _drain_sc():
            pltpu.make_async_copy(comb, acc_sh.at[e0_v.at[1 - slot]],
                                  ssc).wait()
            pltpu.make_async_copy(comb_d, accd_sh.at[e0_v.at[1 - slot]],
                                  ssd).wait()

        @pl.when(t + 1 < nw)
        def _prefetch_idx():
            fire_idx(wid + NW * (t + 1), 1 - slot)

        cg0.wait()
        cg1.wait()
        cg2.wait()

        def pgrp(j, carry2):
            sl = pl.ds(j * LANES, LANES)
            ee = g0_v[sl] + g1_v[sl] + g2_v[sl]
            ex = jnp.exp(ee + ee)
            th = 1.0 - 2.0 / (ex + 1.0)
            p_v[sl] = jnp.exp(th)
            return carry2

        lax.fori_loop(0, CHUNK // LANES, pgrp, 0)
        cr1.wait()
        cr2.wait()

        def rowgrp(g, carry3):
            bs = g * LANES
            pv = p_v[pl.ds(bs, LANES)]
            plsc.store_scatter(comb_d, [bs + iota, jnp.zeros_like(iota)], pv)
            for r in range(LANES):
                pf = jnp.full((LANES,), pv[r], jnp.float32)
                pvec = plsc.pack(pf, pf, format=plsc.PackFormat.INTERLEAVED)
                j = bs + r
                for blk in range(h // (2 * LANES)):
                    sl = pl.ds(blk * 2 * LANES, 2 * LANES)
                    comb[j, sl] = (buf1[j, sl] + buf2[j, sl]) * pvec
            return carry3

        lax.fori_loop(0, CHUNK // LANES, rowgrp, 0)
        pltpu.async_copy(comb, acc_sh.at[e0_v.at[slot]], ssc, add=True)
        pltpu.async_copy(comb_d, accd_sh.at[e0_v.at[slot]], ssd, add=True)
        return carry

    lax.fori_loop(0, nw, chunk_body, 0)
    lslot = lax.rem(nw - 1, 2)
    pltpu.make_async_copy(comb, acc_sh.at[e0_v.at[lslot]], ssc).wait()
    pltpu.make_async_copy(comb_d, accd_sh.at[e0_v.at[lslot]], ssd).wait()
    plsc.subcore_barrier()

    @pl.when(s < 10)
    def _dump_acc():
        pltpu.sync_copy(acc_sh.at[pl.ds(s * zrows, zrows)],
                        partb_hbm.at[c].at[pl.ds(s * zrows, zrows)])
        pltpu.sync_copy(accd_sh.at[pl.ds(s * zrows, zrows)],
                        partd_hbm.at[c].at[pl.ds(s * zrows, zrows)])


def _tc_fin(h_ref, partb_ref, partd_ref, bias_ref, o_ref):
    f32 = jnp.float32
    d = partd_ref[0][:, 0] + partd_ref[1][:, 0]
    p = partb_ref[0].astype(f32) + partb_ref[1].astype(f32)
    h0 = h_ref[0]
    t = jnp.where(d > 0.0, 1.0, 0.0)
    r = 1.0 / jnp.maximum(d, 1e-16)
    o_ref[...] = (h0 * t[:, None] + p * r[:, None]) * (1.0 / 3.0) + bias_ref[...]


def kernel(edge_list, feature_dict, node_type_list, W, b_lin, att_W, att_b, bias):
    l, n, fin = feature_dict.shape
    hdim = W.shape[0]
    e = edge_list.shape[1]
    f32 = jnp.float32

    a1 = att_W[0, :hdim]
    a2 = att_W[0, hdim:]
    c_all = jnp.stack([a1 + a2 / 3.0, a2 / 3.0, a2 / 3.0], 0)

    h_all, s_all = pl.pallas_call(
        _tc_prep,
        grid=(l,),
        in_specs=[
            pl.BlockSpec((1, n, fin), lambda i: (i, 0, 0)),
            pl.BlockSpec((hdim, fin), lambda i: (0, 0)),
            pl.BlockSpec((1, hdim), lambda i: (0, 0)),
            pl.BlockSpec((1, 1, hdim), lambda i: (i, 0, 0)),
            pl.BlockSpec((1, 1), lambda i: (0, 0)),
        ],
        out_specs=[
            pl.BlockSpec((1, n, hdim), lambda i: (i, 0, 0)),
            pl.BlockSpec((1, 1, n), lambda i: (i, 0, 0)),
        ],
        out_shape=[
            jax.ShapeDtypeStruct((l, n, hdim), f32),
            jax.ShapeDtypeStruct((l, 1, n), f32),
        ],
    )(feature_dict, W, b_lin.reshape(1, hdim), c_all.reshape(l, 1, hdim),
      att_b.reshape(1, 1))

    e0 = edge_list[0]
    i1 = edge_list[1] + n
    i2 = edge_list[2] + 2 * n
    hb = h_all.astype(jnp.bfloat16).reshape(l * n, hdim)
    s_flat = s_all.reshape(l * n)

    mesh = plsc.VectorSubcoreMesh(core_axis_name="c", subcore_axis_name="s")
    sc_fn = pl.kernel(
        _sc_edges,
        out_type=[
            jax.ShapeDtypeStruct((NC, n, hdim), jnp.bfloat16),
            jax.ShapeDtypeStruct((NC, n, 8), f32),
        ],
        mesh=mesh,
        compiler_params=pltpu.CompilerParams(needs_layout_passes=False,
                                             use_tc_tiling_on_sc=False),
        scratch_types=[
            pltpu.VMEM((2, CHUNK), jnp.int32),
            pltpu.VMEM((2, CHUNK), jnp.int32),
            pltpu.VMEM((2, CHUNK), jnp.int32),
            pltpu.VMEM((CHUNK,), f32),
            pltpu.VMEM((CHUNK,), f32),
            pltpu.VMEM((CHUNK,), f32),
            pltpu.VMEM((CHUNK,), f32),
            pltpu.VMEM((CHUNK, hdim), jnp.bfloat16),
            pltpu.VMEM((CHUNK, hdim), jnp.bfloat16),
            pltpu.VMEM((CHUNK, hdim), jnp.bfloat16),
            pltpu.VMEM((CHUNK, 8), f32),
            pltpu.VMEM_SHARED((l * n,), f32),
            pltpu.VMEM_SHARED((n, hdim), jnp.bfloat16),
            pltpu.VMEM_SHARED((n, 8), f32),
        ] + [pltpu.SemaphoreType.DMA] * 10,
    )
    partb, partd = sc_fn(e0, i1, i2, hb, s_flat)

    out = pl.pallas_call(
        _tc_fin,
        grid=(1,),
        in_specs=[
            pl.BlockSpec((1, n, hdim), lambda i: (0, 0, 0)),
            pl.BlockSpec((NC, n, hdim), lambda i: (0, 0, 0)),
            pl.BlockSpec((NC, n, 8), lambda i: (0, 0, 0)),
            pl.BlockSpec((1, hdim), lambda i: (0, 0)),
        ],
        out_specs=pl.BlockSpec((n, hdim), lambda i: (0, 0)),
        out_shape=jax.ShapeDtypeStruct((n, hdim), f32),
    )(h_all, partb, partd, bias.reshape(1, hdim))
    return out


# speedup vs baseline: 14.5742x; 1.0373x over previous
import functools

import jax
import jax.numpy as jnp
from jax import lax
from jax.experimental import pallas as pl
from jax.experimental.pallas import tpu as pltpu
from jax.experimental.pallas import tpu_sc as plsc

NC, NS, LANES = 2, 16, 16
NW = NC * NS
CHUNK = 128


def _tc_prep(x_ref, w_ref, b_ref, c_ref, ab_ref, h_ref, hb_ref, s_ref):
    i = pl.program_id(0)
    x = x_ref[0]
    h = jnp.dot(x, w_ref[...].T, preferred_element_type=jnp.float32) + b_ref[...]
    h_ref[0] = h
    hb_ref[0] = h.astype(jnp.bfloat16)
    s = jnp.dot(h, c_ref[0, 0][:, None], preferred_element_type=jnp.float32)[:, 0]
    s_ref[0, 0] = s + jnp.where(i == 0, ab_ref[0, 0], 0.0)


def _sc_edges(e0_hbm, i1_hbm, i2_hbm, hb_hbm, sflat_hbm,
              partb_hbm, partd_hbm,
              e0_v, i1_v, i2_v, s_v, p_v, buf1, buf2, comb, d_v,
              acc_sh,
              se0, se1, se2, sr1, sr2, ssc):
    c = lax.axis_index("c")
    s = lax.axis_index("s")
    wid = s * NC + c
    n3 = sflat_hbm.shape[0]
    n = n3 // 3
    h = hb_hbm.shape[1]
    e = e0_hbm.shape[0]
    total_chunks = e // CHUNK
    base_chunks = total_chunks // NW
    extra = total_chunks - base_chunks * NW
    nw = base_chunks + jnp.where(wid < extra, 1, 0)
    zrows = n // 10
    bf16 = jnp.bfloat16

    pltpu.sync_copy(sflat_hbm, s_v)

    zb = jnp.zeros((2 * LANES,), bf16)
    zf = jnp.zeros((LANES,), jnp.float32)
    iota = lax.iota(jnp.int32, LANES)

    def _zd(j, carry):
        d_v[pl.ds(j * LANES, LANES)] = zf
        return carry

    lax.fori_loop(0, n // LANES, _zd, 0)

    def _zcomb(j, carry):
        for blk in range(h // (2 * LANES)):
            comb[0, j, pl.ds(blk * 2 * LANES, 2 * LANES)] = zb
        return carry

    lax.fori_loop(0, CHUNK, _zcomb, 0)

    zpc = CHUNK - 3

    @pl.when(s < 10)
    def _zero_acc():
        def _zcp(k, carry):
            r0 = s * zrows + k * zpc
            pltpu.sync_copy(comb.at[0].at[pl.ds(0, zpc)],
                            acc_sh.at[pl.ds(r0, zpc)])
            return carry

        lax.fori_loop(0, zrows // zpc, _zcp, 0)

    plsc.subcore_barrier()

    def fire_idx(chunk_id, slot):
        off = chunk_id * CHUNK
        pltpu.async_copy(e0_hbm.at[pl.ds(off, CHUNK)], e0_v.at[slot], se0)
        pltpu.async_copy(i1_hbm.at[pl.ds(off, CHUNK)], i1_v.at[slot], se1)
        pltpu.async_copy(i2_hbm.at[pl.ds(off, CHUNK)], i2_v.at[slot], se2)

    fire_idx(wid, 0)

    def chunk_body(t, carry):
        islot = lax.rem(t, 3)
        cslot = lax.rem(t, 2)
        pltpu.make_async_copy(
            e0_hbm.at[pl.ds(0, CHUNK)], e0_v.at[islot], se0).wait()
        pltpu.make_async_copy(
            i1_hbm.at[pl.ds(0, CHUNK)], i1_v.at[islot], se1).wait()
        pltpu.make_async_copy(
            i2_hbm.at[pl.ds(0, CHUNK)], i2_v.at[islot], se2).wait()
        cr1 = pltpu.async_copy(hb_hbm.at[i1_v.at[islot]], buf1, sr1)
        cr2 = pltpu.async_copy(hb_hbm.at[i2_v.at[islot]], buf2, sr2)

        @pl.when(t > 1)
        def _drain_sc():
            pltpu.make_async_copy(comb.at[cslot], acc_sh.at[e0_v.at[islot]],
                                  ssc).wait()

        @pl.when(t + 1 < nw)
        def _prefetch_idx():
            fire_idx(wid + NW * (t + 1), lax.rem(t + 1, 3))

        def pgrp(j, carry2):
            sl = pl.ds(j * LANES, LANES)
            idx0 = e0_v[islot, sl]
            g0 = plsc.load_gather(s_v, [idx0])
            g1 = plsc.load_gather(s_v, [i1_v[islot, sl]])
            g2 = plsc.load_gather(s_v, [i2_v[islot, sl]])
            ee = g0 + g1 + g2
            ex = jnp.exp(ee + ee)
            th = 1.0 - 2.0 / (ex + 1.0)
            p = jnp.exp(th)
            p_v[sl] = p
            for r in range(LANES):
                plsc.addupdate_scatter(d_v, [idx0], p, mask=iota == r)
            return carry2

        lax.fori_loop(0, CHUNK // LANES, pgrp, 0)
        cr1.wait()
        cr2.wait()

        def rowgrp(g, carry3):
            bs = g * LANES
            pv = p_v[pl.ds(bs, LANES)]
            for r in range(LANES):
                pf = jnp.full((LANES,), pv[r], jnp.float32)
                pvec = plsc.pack(pf, pf, format=plsc.PackFormat.INTERLEAVED)
                j = bs + r
                for blk in range(h // (2 * LANES)):
                    sl = pl.ds(blk * 2 * LANES, 2 * LANES)
                    comb[cslot, j, sl] = (buf1[j, sl] + buf2[j, sl]) * pvec
            return carry3

        lax.fori_loop(0, CHUNK // LANES, rowgrp, 0)
        pltpu.async_copy(comb.at[cslot], acc_sh.at[e0_v.at[islot]], ssc,
                         add=True)
        return carry

    lax.fori_loop(0, nw, chunk_body, 0)
    pltpu.make_async_copy(comb.at[0], acc_sh.at[e0_v.at[0]], ssc).wait()
    pltpu.make_async_copy(comb.at[0], acc_sh.at[e0_v.at[0]], ssc).wait()
    pltpu.sync_copy(d_v, partd_hbm.at[wid])
    plsc.subcore_barrier()

    @pl.when(s < 10)
    def _dump_acc():
        pltpu.sync_copy(acc_sh.at[pl.ds(s * zrows, zrows)],
                        partb_hbm.at[c].at[pl.ds(s * zrows, zrows)])


def _tc_fin(h_ref, partb_ref, partd_ref, bias_ref, o_ref):
    f32 = jnp.float32
    d = jnp.sum(partd_ref[...], axis=0)
    p = partb_ref[0].astype(f32) + partb_ref[1].astype(f32)
    h0 = h_ref[0]
    t = jnp.where(d > 0.0, 1.0, 0.0)
    r = 1.0 / jnp.maximum(d, 1e-16)
    o_ref[...] = (h0 * t[:, None] + p * r[:, None]) * (1.0 / 3.0) + bias_ref[...]


def kernel(edge_list, feature_dict, node_type_list, W, b_lin, att_W, att_b, bias):
    l, n, fin = feature_dict.shape
    hdim = W.shape[0]
    e = edge_list.shape[1]
    f32 = jnp.float32

    a1 = att_W[0, :hdim]
    a2 = att_W[0, hdim:]
    c_all = jnp.stack([a1 + a2 / 3.0, a2 / 3.0, a2 / 3.0], 0)

    h_all, hb_all, s_all = pl.pallas_call(
        _tc_prep,
        grid=(l,),
        in_specs=[
            pl.BlockSpec((1, n, fin), lambda i: (i, 0, 0)),
            pl.BlockSpec((hdim, fin), lambda i: (0, 0)),
            pl.BlockSpec((1, hdim), lambda i: (0, 0)),
            pl.BlockSpec((1, 1, hdim), lambda i: (i, 0, 0)),
            pl.BlockSpec((1, 1), lambda i: (0, 0)),
        ],
        out_specs=[
            pl.BlockSpec((1, n, hdim), lambda i: (i, 0, 0)),
            pl.BlockSpec((1, n, hdim), lambda i: (i, 0, 0)),
            pl.BlockSpec((1, 1, n), lambda i: (i, 0, 0)),
        ],
        out_shape=[
            jax.ShapeDtypeStruct((l, n, hdim), f32),
            jax.ShapeDtypeStruct((l, n, hdim), jnp.bfloat16),
            jax.ShapeDtypeStruct((l, 1, n), f32),
        ],
    )(feature_dict, W, b_lin.reshape(1, hdim), c_all.reshape(l, 1, hdim),
      att_b.reshape(1, 1))

    e0 = edge_list[0]
    i1 = edge_list[1] + n
    i2 = edge_list[2] + 2 * n
    hb = hb_all.reshape(l * n, hdim)
    s_flat = s_all.reshape(l * n)

    mesh = plsc.VectorSubcoreMesh(core_axis_name="c", subcore_axis_name="s")
    sc_fn = pl.kernel(
        _sc_edges,
        out_type=[
            jax.ShapeDtypeStruct((NC, n, hdim), jnp.bfloat16),
            jax.ShapeDtypeStruct((NW, n), f32),
        ],
        mesh=mesh,
        compiler_params=pltpu.CompilerParams(needs_layout_passes=False,
                                             use_tc_tiling_on_sc=False),
        scratch_types=[
            pltpu.VMEM((3, CHUNK), jnp.int32),
            pltpu.VMEM((3, CHUNK), jnp.int32),
            pltpu.VMEM((3, CHUNK), jnp.int32),
            pltpu.VMEM((l * n,), f32),
            pltpu.VMEM((CHUNK,), f32),
            pltpu.VMEM((CHUNK, hdim), jnp.bfloat16),
            pltpu.VMEM((CHUNK, hdim), jnp.bfloat16),
            pltpu.VMEM((2, CHUNK, hdim), jnp.bfloat16),
            pltpu.VMEM((n,), f32),
            pltpu.VMEM_SHARED((n, hdim), jnp.bfloat16),
        ] + [pltpu.SemaphoreType.DMA] * 6,
    )
    partb, partd = sc_fn(e0, i1, i2, hb, s_flat)

    out = pl.pallas_call(
        _tc_fin,
        grid=(1,),
        in_specs=[
            pl.BlockSpec((1, n, hdim), lambda i: (0, 0, 0)),
            pl.BlockSpec((NC, n, hdim), lambda i: (0, 0, 0)),
            pl.BlockSpec((NW, n), lambda i: (0, 0)),
            pl.BlockSpec((1, hdim), lambda i: (0, 0)),
        ],
        out_specs=pl.BlockSpec((n, hdim), lambda i: (0, 0)),
        out_shape=jax.ShapeDtypeStruct((n, hdim), f32),
    )(h_all, partb, partd, bias.reshape(1, hdim))
    return out


# speedup vs baseline: 15.7772x; 1.0825x over previous
import functools

import jax
import jax.numpy as jnp
from jax import lax
from jax.experimental import pallas as pl
from jax.experimental.pallas import tpu as pltpu
from jax.experimental.pallas import tpu_sc as plsc

NC, NS, LANES = 2, 16, 16
NW = NC * NS
CHUNK = 128


def _tc_prep(x_ref, w_ref, b_ref, c_ref, ab_ref, h_ref, hb_ref, s_ref):
    i = pl.program_id(0)
    x = x_ref[0]
    h = jnp.dot(x, w_ref[...].T, preferred_element_type=jnp.float32) + b_ref[...]
    h_ref[0] = h
    hb_ref[0] = h.astype(jnp.bfloat16)
    s = jnp.dot(h, c_ref[0, 0][:, None], preferred_element_type=jnp.float32)[:, 0]
    s_ref[0, 0] = s + jnp.where(i == 0, ab_ref[0, 0], 0.0)


def _sc_edges(e0_hbm, i1_hbm, i2_hbm, hb_hbm, sflat_hbm,
              partb_hbm, partd_hbm,
              e0_v, i1_v, i2_v, s_v, p_v, buf1, buf2, comb, d_v,
              acc_sh,
              se0, se1, se2, sr1, sr2, ssc):
    c = lax.axis_index("c")
    s = lax.axis_index("s")
    wid = s * NC + c
    n3 = sflat_hbm.shape[0]
    n = n3 // 3
    h = hb_hbm.shape[1]
    e = e0_hbm.shape[0]
    total_chunks = e // CHUNK
    base_chunks = total_chunks // NW
    extra = total_chunks - base_chunks * NW
    nw = base_chunks + jnp.where(wid < extra, 1, 0)
    zrows = n // 10
    bf16 = jnp.bfloat16

    pltpu.sync_copy(sflat_hbm, s_v)

    zb = jnp.zeros((2 * LANES,), bf16)
    zf = jnp.zeros((LANES,), jnp.float32)
    iota = lax.iota(jnp.int32, LANES)

    def _zd(j, carry):
        d_v[pl.ds(j * LANES, LANES)] = zf
        return carry

    lax.fori_loop(0, n // LANES, _zd, 0)

    def _zcomb(j, carry):
        for blk in range(h // (2 * LANES)):
            comb[j, pl.ds(blk * 2 * LANES, 2 * LANES)] = zb
        return carry

    lax.fori_loop(0, CHUNK, _zcomb, 0)

    zpc = CHUNK - 3

    @pl.when(s < 10)
    def _zero_acc():
        def _zcp(k, carry):
            r0 = s * zrows + k * zpc
            pltpu.sync_copy(comb.at[pl.ds(0, zpc)],
                            acc_sh.at[pl.ds(r0, zpc)])
            return carry

        lax.fori_loop(0, zrows // zpc, _zcp, 0)

    plsc.subcore_barrier()

    def fire_idx(chunk_id, slot):
        off = chunk_id * CHUNK
        pltpu.async_copy(e0_hbm.at[pl.ds(off, CHUNK)], e0_v.at[slot], se0)
        pltpu.async_copy(i1_hbm.at[pl.ds(off, CHUNK)], i1_v.at[slot], se1)
        pltpu.async_copy(i2_hbm.at[pl.ds(off, CHUNK)], i2_v.at[slot], se2)

    def fire_rows(islot, bslot):
        pltpu.async_copy(hb_hbm.at[i1_v.at[islot]], buf1.at[bslot], sr1)
        pltpu.async_copy(hb_hbm.at[i2_v.at[islot]], buf2.at[bslot], sr2)

    def wait_idx(islot):
        pltpu.make_async_copy(
            e0_hbm.at[pl.ds(0, CHUNK)], e0_v.at[islot], se0).wait()
        pltpu.make_async_copy(
            i1_hbm.at[pl.ds(0, CHUNK)], i1_v.at[islot], se1).wait()
        pltpu.make_async_copy(
            i2_hbm.at[pl.ds(0, CHUNK)], i2_v.at[islot], se2).wait()

    def wait_rows(islot, bslot):
        pltpu.make_async_copy(hb_hbm.at[i1_v.at[islot]], buf1.at[bslot],
                              sr1).wait()
        pltpu.make_async_copy(hb_hbm.at[i2_v.at[islot]], buf2.at[bslot],
                              sr2).wait()

    fire_idx(wid, 0)
    wait_idx(0)
    fire_rows(0, 0)
    fire_idx(wid + NW, 1)

    def chunk_body(t, carry):
        islot = lax.rem(t, 3)
        bslot = lax.rem(t, 2)
        wait_rows(islot, bslot)

        @pl.when(t + 1 < nw)
        def _prefetch_rows():
            wait_idx(lax.rem(t + 1, 3))
            fire_rows(lax.rem(t + 1, 3), 1 - bslot)

        @pl.when(t > 0)
        def _drain_sc():
            pltpu.make_async_copy(comb, acc_sh.at[e0_v.at[islot]],
                                  ssc).wait()

        @pl.when(t + 2 < nw)
        def _prefetch_idx():
            fire_idx(wid + NW * (t + 2), lax.rem(t + 2, 3))

        def pgrp(j, carry2):
            sl = pl.ds(j * LANES, LANES)
            idx0 = e0_v[islot, sl]
            g0 = plsc.load_gather(s_v, [idx0])
            g1 = plsc.load_gather(s_v, [i1_v[islot, sl]])
            g2 = plsc.load_gather(s_v, [i2_v[islot, sl]])
            ee = g0 + g1 + g2
            ex = jnp.exp(ee + ee)
            th = 1.0 - 2.0 / (ex + 1.0)
            p = jnp.exp(th)
            p_v[sl] = p
            for r in range(LANES):
                plsc.addupdate_scatter(d_v, [idx0], p, mask=iota == r)
            return carry2

        lax.fori_loop(0, CHUNK // LANES, pgrp, 0)

        def rowgrp(g, carry3):
            bs = g * LANES
            pv = p_v[pl.ds(bs, LANES)]
            for r in range(LANES):
                pf = jnp.full((LANES,), pv[r], jnp.float32)
                pvec = plsc.pack(pf, pf, format=plsc.PackFormat.INTERLEAVED)
                j = bs + r
                for blk in range(h // (2 * LANES)):
                    sl = pl.ds(blk * 2 * LANES, 2 * LANES)
                    comb[j, sl] = (buf1[bslot, j, sl] + buf2[bslot, j, sl]) * pvec
            return carry3

        lax.fori_loop(0, CHUNK // LANES, rowgrp, 0)
        pltpu.async_copy(comb, acc_sh.at[e0_v.at[islot]], ssc, add=True)
        return carry

    lax.fori_loop(0, nw, chunk_body, 0)
    pltpu.make_async_copy(comb, acc_sh.at[e0_v.at[0]], ssc).wait()
    pltpu.sync_copy(d_v, partd_hbm.at[wid])
    plsc.subcore_barrier()

    @pl.when(s < 10)
    def _dump_acc():
        pltpu.sync_copy(acc_sh.at[pl.ds(s * zrows, zrows)],
                        partb_hbm.at[c].at[pl.ds(s * zrows, zrows)])


def _tc_fin(h_ref, partb_ref, partd_ref, bias_ref, o_ref):
    f32 = jnp.float32
    d = jnp.sum(partd_ref[...], axis=0)
    p = partb_ref[0].astype(f32) + partb_ref[1].astype(f32)
    h0 = h_ref[0]
    t = jnp.where(d > 0.0, 1.0, 0.0)
    r = 1.0 / jnp.maximum(d, 1e-16)
    o_ref[...] = (h0 * t[:, None] + p * r[:, None]) * (1.0 / 3.0) + bias_ref[...]


def kernel(edge_list, feature_dict, node_type_list, W, b_lin, att_W, att_b, bias):
    l, n, fin = feature_dict.shape
    hdim = W.shape[0]
    e = edge_list.shape[1]
    f32 = jnp.float32

    a1 = att_W[0, :hdim]
    a2 = att_W[0, hdim:]
    c_all = jnp.stack([a1 + a2 / 3.0, a2 / 3.0, a2 / 3.0], 0)

    h_all, hb_all, s_all = pl.pallas_call(
        _tc_prep,
        grid=(l,),
        in_specs=[
            pl.BlockSpec((1, n, fin), lambda i: (i, 0, 0)),
            pl.BlockSpec((hdim, fin), lambda i: (0, 0)),
            pl.BlockSpec((1, hdim), lambda i: (0, 0)),
            pl.BlockSpec((1, 1, hdim), lambda i: (i, 0, 0)),
            pl.BlockSpec((1, 1), lambda i: (0, 0)),
        ],
        out_specs=[
            pl.BlockSpec((1, n, hdim), lambda i: (i, 0, 0)),
            pl.BlockSpec((1, n, hdim), lambda i: (i, 0, 0)),
            pl.BlockSpec((1, 1, n), lambda i: (i, 0, 0)),
        ],
        out_shape=[
            jax.ShapeDtypeStruct((l, n, hdim), f32),
            jax.ShapeDtypeStruct((l, n, hdim), jnp.bfloat16),
            jax.ShapeDtypeStruct((l, 1, n), f32),
        ],
    )(feature_dict, W, b_lin.reshape(1, hdim), c_all.reshape(l, 1, hdim),
      att_b.reshape(1, 1))

    e0 = edge_list[0]
    i1 = edge_list[1] + n
    i2 = edge_list[2] + 2 * n
    hb = hb_all.reshape(l * n, hdim)
    s_flat = s_all.reshape(l * n)

    mesh = plsc.VectorSubcoreMesh(core_axis_name="c", subcore_axis_name="s")
    sc_fn = pl.kernel(
        _sc_edges,
        out_type=[
            jax.ShapeDtypeStruct((NC, n, hdim), jnp.bfloat16),
            jax.ShapeDtypeStruct((NW, n), f32),
        ],
        mesh=mesh,
        compiler_params=pltpu.CompilerParams(needs_layout_passes=False,
                                             use_tc_tiling_on_sc=False),
        scratch_types=[
            pltpu.VMEM((3, CHUNK), jnp.int32),
            pltpu.VMEM((3, CHUNK), jnp.int32),
            pltpu.VMEM((3, CHUNK), jnp.int32),
            pltpu.VMEM((l * n,), f32),
            pltpu.VMEM((CHUNK,), f32),
            pltpu.VMEM((2, CHUNK, hdim), jnp.bfloat16),
            pltpu.VMEM((2, CHUNK, hdim), jnp.bfloat16),
            pltpu.VMEM((CHUNK, hdim), jnp.bfloat16),
            pltpu.VMEM((n,), f32),
            pltpu.VMEM_SHARED((n, hdim), jnp.bfloat16),
        ] + [pltpu.SemaphoreType.DMA] * 6,
    )
    partb, partd = sc_fn(e0, i1, i2, hb, s_flat)

    out = pl.pallas_call(
        _tc_fin,
        grid=(1,),
        in_specs=[
            pl.BlockSpec((1, n, hdim), lambda i: (0, 0, 0)),
            pl.BlockSpec((NC, n, hdim), lambda i: (0, 0, 0)),
            pl.BlockSpec((NW, n), lambda i: (0, 0)),
            pl.BlockSpec((1, hdim), lambda i: (0, 0)),
        ],
        out_specs=pl.BlockSpec((n, hdim), lambda i: (0, 0)),
        out_shape=jax.ShapeDtypeStruct((n, hdim), f32),
    )(h_all, partb, partd, bias.reshape(1, hdim))
    return out
